# baseline (speedup 1.0000x reference)
import functools

import jax
import jax.numpy as jnp
import numpy as np
from jax.experimental import pallas as pl
from jax.experimental.pallas import tpu as pltpu

N_NODES = 50000


def _mm_kernel(x_ref, w_ref, b_ref, o_ref):
    o_ref[...] = (
        jnp.dot(x_ref[...], w_ref[...], preferred_element_type=jnp.float32)
        + b_ref[...]
    )


def _dense(x, W, b, block=1024):
    n, din = x.shape
    m = W.shape[1]
    npad = ((n + block - 1) // block) * block
    if npad != n:
        x = jnp.pad(x, ((0, npad - n), (0, 0)))
    out = pl.pallas_call(
        _mm_kernel,
        grid=(npad // block,),
        in_specs=[
            pl.BlockSpec((block, din), lambda i: (i, 0)),
            pl.BlockSpec((din, m), lambda i: (0, 0)),
            pl.BlockSpec((m,), lambda i: (0,)),
        ],
        out_specs=pl.BlockSpec((block, m), lambda i: (i, 0)),
        out_shape=jax.ShapeDtypeStruct((npad, m), jnp.float32),
    )(x, W, b)
    return out[:n]


def _tconv(x, edge_index, Wq, bq, Wk, bk, Wv, bv, Ws, bs, heads, C):
    n = x.shape[0]
    src = edge_index[0]
    dst = edge_index[1]
    W = jnp.concatenate([Wq, Wk, Wv, Ws], axis=1)
    b = jnp.concatenate([bq, bk, bv, bs], axis=0)
    hc = Wq.shape[1]
    qkvs = _dense(x, W, b)
    q = qkvs[:, :hc].reshape(n, heads, C)
    k = qkvs[:, hc:2 * hc].reshape(n, heads, C)
    v = qkvs[:, 2 * hc:3 * hc].reshape(n, heads, C)
    skip = qkvs[:, 3 * hc:]
    alpha = jnp.sum(q[dst] * k[src], axis=-1) / float(np.sqrt(C))
    amax = jax.ops.segment_max(alpha, dst, num_segments=n)
    amax = jnp.where(jnp.isfinite(amax), amax, 0.0)
    ex = jnp.exp(alpha - amax[dst])
    denom = jax.ops.segment_sum(ex, dst, num_segments=n)
    a = ex / (denom[dst] + 1e-16)
    msg = v[src] * a[:, :, None]
    out = jax.ops.segment_sum(msg, dst, num_segments=n).reshape(n, heads * C)
    return out + skip


def kernel(x, edge_index, Wq1, Wk1, Wv1, Ws1, bq1, bk1, bv1, bs1, Wq2, Wk2, Wv2, Ws2, bq2, bk2, bv2, bs2, Wq3, Wk3, Wv3, Ws3, bq3, bk3, bv3, bs3):
    h = _tconv(x, edge_index, Wq1, bq1, Wk1, bk1, Wv1, bv1, Ws1, bs1, 4, 50)
    h = jax.nn.leaky_relu(h, 0.1)
    h = _tconv(h, edge_index, Wq2, bq2, Wk2, bk2, Wv2, bv2, Ws2, bs2, 4, 25)
    h = jax.nn.leaky_relu(h, 0.1)
    h = _tconv(h, edge_index, Wq3, bq3, Wk3, bk3, Wv3, bv3, Ws3, bs3, 4, 10)
    return jax.nn.log_softmax(h, axis=1)


# speedup vs baseline: 1.0001x; 1.0001x over previous
import jax
import jax.numpy as jnp
import numpy as np
from jax.experimental import pallas as pl


def _mm_kernel(x_ref, w_ref, b_ref, o_ref):
    o_ref[...] = (
        jnp.dot(x_ref[...], w_ref[...], preferred_element_type=jnp.float32)
        + b_ref[...]
    )


def _dense(x, W, b, block=1024):
    n, din = x.shape
    m = W.shape[1]
    npad = ((n + block - 1) // block) * block
    if npad != n:
        x = jnp.pad(x, ((0, npad - n), (0, 0)))
    out = pl.pallas_call(
        _mm_kernel,
        grid=(npad // block,),
        in_specs=[
            pl.BlockSpec((block, din), lambda i: (i, 0)),
            pl.BlockSpec((din, m), lambda i: (0, 0)),
            pl.BlockSpec((m,), lambda i: (0,)),
        ],
        out_specs=pl.BlockSpec((block, m), lambda i: (i, 0)),
        out_shape=jax.ShapeDtypeStruct((npad, m), jnp.float32),
    )(x, W, b)
    return out[:n]


def _tconv(x, edge_index, Wq, bq, Wk, bk, Wv, bv, Ws, bs, heads, C):
    n = x.shape[0]
    src = edge_index[0]
    dst = edge_index[1]
    W = jnp.concatenate([Wq, Wk, Wv, Ws], axis=1)
    b = jnp.concatenate([bq, bk, bv, bs], axis=0)
    hc = Wq.shape[1]
    qkvs = _dense(x, W, b)
    q = qkvs[:, :hc].reshape(n, heads, C)
    k = qkvs[:, hc:2 * hc].reshape(n, heads, C)
    v = qkvs[:, 2 * hc:3 * hc].reshape(n, heads, C)
    skip = qkvs[:, 3 * hc:]
    alpha = jnp.sum(q[dst] * k[src], axis=-1) / float(np.sqrt(C))
    amax = jax.ops.segment_max(alpha, dst, num_segments=n)
    amax = jnp.where(jnp.isfinite(amax), amax, 0.0)
    ex = jnp.exp(alpha - amax[dst])
    denom = jax.ops.segment_sum(ex, dst, num_segments=n)
    a = ex / (denom[dst] + 1e-16)
    msg = v[src] * a[:, :, None]
    out = jax.ops.segment_sum(msg, dst, num_segments=n).reshape(n, heads * C)
    return out + skip


def kernel(x, edge_index, Wq1, Wk1, Wv1, Ws1, bq1, bk1, bv1, bs1, Wq2, Wk2, Wv2, Ws2, bq2, bk2, bv2, bs2, Wq3, Wk3, Wv3, Ws3, bq3, bk3, bv3, bs3):
    h = _tconv(x, edge_index, Wq1, bq1, Wk1, bk1, Wv1, bv1, Ws1, bs1, 4, 50)
    h = jax.nn.leaky_relu(h, 0.1)
    h = _tconv(h, edge_index, Wq2, bq2, Wk2, bk2, Wv2, bv2, Ws2, bs2, 4, 25)
    h = jax.nn.leaky_relu(h, 0.1)
    h = _tconv(h, edge_index, Wq3, bq3, Wk3, bk3, Wv3, bv3, Ws3, bs3, 4, 10)
    return jax.nn.log_softmax(h, axis=1)
